# Initial kernel scaffold; baseline (speedup 1.0000x reference)
import jax
import jax.numpy as jnp
from jax.experimental import pallas as pl


def kernel(h, x, edge_index, edge_attr, W_in, b_in, We1, be1, We2, be2, Wx1, bx1, Wx2, bx2, Wh1, bh1, Wh2, bh2, W_out, b_out, W_fin, b_fin):
    raise NotImplementedError("write your pallas kernel here")


# speedup vs baseline: 2.1390x; 2.1390x over previous
import functools

import jax
import jax.numpy as jnp
from jax.experimental import pallas as pl
from jax.experimental.pallas import tpu as pltpu
from jax.experimental.pallas import tpu_sc as plsc

N = 10000
E = 320000
H = 128
DE = 16
AVG_DEG = float(E) / float(N)

NC = 2
NS = 16
NW = NC * NS
CHUNK = 128
NCHUNK = E // CHUNK
FULL_ROUNDS = NCHUNK // NW
TAIL = NCHUNK - FULL_ROUNDS * NW

BN = 1000
BE = 1280
NPT = 624
NREM = N - NS * NPT

_f32 = jnp.float32


def _silu(v):
    return v * jax.nn.sigmoid(v)


def _mesh():
    return plsc.VectorSubcoreMesh(core_axis_name="c", subcore_axis_name="s",
                                  num_cores=NC, num_subcores=NS)


def _prep_body(h_ref, xp_ref, win_ref, bin_ref, a_ref, b_ref,
               h1_ref, ts_ref, td_ref):
    h1 = jnp.dot(h_ref[...], win_ref[...],
                 preferred_element_type=_f32) + bin_ref[...]
    h1_ref[...] = h1
    xp = xp_ref[...]
    p = jnp.dot(h1, a_ref[...], preferred_element_type=_f32)
    q = jnp.dot(h1, b_ref[...], preferred_element_type=_f32)
    ts_ref[...] = jnp.stack([p, xp], axis=1)
    td_ref[...] = jnp.stack([q, xp], axis=1)


def _tc_prep(h, xpad, w_in, b_in, a0, b0):
    grid = (N // BN,)
    blk = lambda i: (i, 0)
    blk3 = lambda i: (i, 0, 0)
    full = lambda i: (0, 0)
    return pl.pallas_call(
        _prep_body,
        grid=grid,
        in_specs=[
            pl.BlockSpec((BN, H), blk),
            pl.BlockSpec((BN, H), blk),
            pl.BlockSpec((H, H), full),
            pl.BlockSpec((1, H), full),
            pl.BlockSpec((H, H), full),
            pl.BlockSpec((H, H), full),
        ],
        out_specs=[
            pl.BlockSpec((BN, H), blk),
            pl.BlockSpec((BN, 2, H), blk3),
            pl.BlockSpec((BN, 2, H), blk3),
        ],
        out_shape=[
            jax.ShapeDtypeStruct((N, H), _f32),
            jax.ShapeDtypeStruct((N, 2, H), _f32),
            jax.ShapeDtypeStruct((N, 2, H), _f32),
        ],
        compiler_params=pltpu.CompilerParams(
            dimension_semantics=("arbitrary",)),
    )(h, xpad, w_in, b_in, a0, b0)


def _edge0_body(gs_ref, gd_ref, ea_ref, wr_ref, wea_ref, be1_ref,
                we2_ref, be2_ref, wx1_ref, bx1_ref, wx2_ref, bx2_ref,
                m_ref, t_ref):
    ps = gs_ref[:, 0, :]
    qd = gd_ref[:, 0, :]
    diff = gs_ref[:, 1, :] - gd_ref[:, 1, :]
    r2 = jnp.sum(diff * diff, axis=1, keepdims=True)
    a = (ps + qd + r2 * wr_ref[...] +
         jnp.dot(ea_ref[...], wea_ref[...], preferred_element_type=_f32) +
         be1_ref[...])
    m = _silu(jnp.dot(_silu(a), we2_ref[...],
                      preferred_element_type=_f32) + be2_ref[...])
    u = _silu(jnp.dot(m, wx1_ref[...],
                      preferred_element_type=_f32) + bx1_ref[...])
    w = jnp.sum(u * wx2_ref[...], axis=1, keepdims=True) + bx2_ref[...]
    m_ref[...] = m
    t_ref[...] = diff * w


def _tc_edge0(gs, gd, ea, wr, wea, be1, we2, be2, wx1, bx1, wx2r, bx2):
    grid = (E // BE,)
    blk = lambda i: (i, 0)
    blk3 = lambda i: (i, 0, 0)
    full = lambda i: (0, 0)
    return pl.pallas_call(
        _edge0_body,
        grid=grid,
        in_specs=[
            pl.BlockSpec((BE, 2, H), blk3),
            pl.BlockSpec((BE, 2, H), blk3),
            pl.BlockSpec((BE, DE), blk),
            pl.BlockSpec((1, H), full),
            pl.BlockSpec((DE, H), full),
            pl.BlockSpec((1, H), full),
            pl.BlockSpec((H, H), full),
            pl.BlockSpec((1, H), full),
            pl.BlockSpec((H, H), full),
            pl.BlockSpec((1, H), full),
            pl.BlockSpec((1, H), full),
            pl.BlockSpec((1, 1), full),
        ],
        out_specs=[
            pl.BlockSpec((BE, H), blk),
            pl.BlockSpec((BE, H), blk),
        ],
        out_shape=[
            jax.ShapeDtypeStruct((E, H), _f32),
            jax.ShapeDtypeStruct((E, H), _f32),
        ],
        compiler_params=pltpu.CompilerParams(
            dimension_semantics=("arbitrary",)),
    )(gs, gd, ea, wr, wea, be1, we2, be2, wx1, bx1, wx2r, bx2)


def _edge1_body(gs_ref, gd_ref, ea_ref, wr_ref, wea_ref, be1_ref,
                we2_ref, be2_ref, m_ref):
    diff = gs_ref[:, 1, :] - gd_ref[:, 1, :]
    r2 = jnp.sum(diff * diff, axis=1, keepdims=True)
    a = (gs_ref[:, 0, :] + gd_ref[:, 0, :] + r2 * wr_ref[...] +
         jnp.dot(ea_ref[...], wea_ref[...], preferred_element_type=_f32) +
         be1_ref[...])
    m_ref[...] = _silu(jnp.dot(_silu(a), we2_ref[...],
                               preferred_element_type=_f32) + be2_ref[...])


def _tc_edge1(gs, gd, ea, wr, wea, be1, we2, be2):
    grid = (E // BE,)
    blk = lambda i: (i, 0)
    blk3 = lambda i: (i, 0, 0)
    full = lambda i: (0, 0)
    return pl.pallas_call(
        _edge1_body,
        grid=grid,
        in_specs=[
            pl.BlockSpec((BE, 2, H), blk3),
            pl.BlockSpec((BE, 2, H), blk3),
            pl.BlockSpec((BE, DE), blk),
            pl.BlockSpec((1, H), full),
            pl.BlockSpec((DE, H), full),
            pl.BlockSpec((1, H), full),
            pl.BlockSpec((H, H), full),
            pl.BlockSpec((1, H), full),
        ],
        out_specs=[pl.BlockSpec((BE, H), blk)],
        out_shape=[jax.ShapeDtypeStruct((E, H), _f32)],
        compiler_params=pltpu.CompilerParams(
            dimension_semantics=("arbitrary",)),
    )(gs, gd, ea, wr, wea, be1, we2, be2)[0]


def _node0_body(h1_ref, xp_ref, aggh_ref, aggx_ref, wh1a_ref, wh1b_ref,
                bh1_ref, wh2_ref, bh2_ref, a1_ref, b1_ref,
                h2_ref, ts_ref, td_ref):
    h1 = h1_ref[...]
    aggh = aggh_ref[0] + aggh_ref[1]
    pre = (jnp.dot(h1, wh1a_ref[...], preferred_element_type=_f32) +
           jnp.dot(aggh, wh1b_ref[...], preferred_element_type=_f32) +
           bh1_ref[...])
    upd = jnp.dot(_silu(pre), wh2_ref[...],
                  preferred_element_type=_f32) + bh2_ref[...]
    h2 = h1 + upd
    h2_ref[...] = h2
    xn = xp_ref[...] + (aggx_ref[0] + aggx_ref[1]) * (1.0 / AVG_DEG)
    p = jnp.dot(h2, a1_ref[...], preferred_element_type=_f32)
    q = jnp.dot(h2, b1_ref[...], preferred_element_type=_f32)
    ts_ref[...] = jnp.stack([p, xn], axis=1)
    td_ref[...] = jnp.stack([q, xn], axis=1)


def _tc_node0(h1, xpad, aggh, aggx, wh1a, wh1b, bh1, wh2, bh2, a1, b1):
    grid = (N // BN,)
    blk = lambda i: (i, 0)
    blk3 = lambda i: (i, 0, 0)
    blk3c = lambda i: (0, i, 0)
    full = lambda i: (0, 0)
    return pl.pallas_call(
        _node0_body,
        grid=grid,
        in_specs=[
            pl.BlockSpec((BN, H), blk),
            pl.BlockSpec((BN, H), blk),
            pl.BlockSpec((NC, BN, H), blk3c),
            pl.BlockSpec((NC, BN, H), blk3c),
            pl.BlockSpec((H, H), full),
            pl.BlockSpec((H, H), full),
            pl.BlockSpec((1, H), full),
            pl.BlockSpec((H, H), full),
            pl.BlockSpec((1, H), full),
            pl.BlockSpec((H, H), full),
            pl.BlockSpec((H, H), full),
        ],
        out_specs=[
            pl.BlockSpec((BN, H), blk),
            pl.BlockSpec((BN, 2, H), blk3),
            pl.BlockSpec((BN, 2, H), blk3),
        ],
        out_shape=[
            jax.ShapeDtypeStruct((N, H), _f32),
            jax.ShapeDtypeStruct((N, 2, H), _f32),
            jax.ShapeDtypeStruct((N, 2, H), _f32),
        ],
        compiler_params=pltpu.CompilerParams(
            dimension_semantics=("arbitrary",)),
    )(h1, xpad, aggh, aggx, wh1a, wh1b, bh1, wh2, bh2, a1, b1)


def _node1_body(h2_ref, aggh_ref, wh1a_ref, wh1b_ref, bh1_ref, wh2_ref,
                bh2_ref, wout_ref, bout_ref, wfin_ref, bfin_ref, out_ref):
    h2 = h2_ref[...]
    aggh = aggh_ref[0] + aggh_ref[1]
    pre = (jnp.dot(h2, wh1a_ref[...], preferred_element_type=_f32) +
           jnp.dot(aggh, wh1b_ref[...], preferred_element_type=_f32) +
           bh1_ref[...])
    upd = jnp.dot(_silu(pre), wh2_ref[...],
                  preferred_element_type=_f32) + bh2_ref[...]
    hf = h2 + upd
    ho = jnp.dot(hf, wout_ref[...], preferred_element_type=_f32) + bout_ref[...]
    out_ref[...] = jnp.sum(ho * wfin_ref[...], axis=1,
                           keepdims=True) + bfin_ref[...]


def _tc_node1(h2, aggh, wh1a, wh1b, bh1, wh2, bh2, wout, bout, wfinr, bfin):
    grid = (N // BN,)
    blk = lambda i: (i, 0)
    blk3c = lambda i: (0, i, 0)
    full = lambda i: (0, 0)
    return pl.pallas_call(
        _node1_body,
        grid=grid,
        in_specs=[
            pl.BlockSpec((BN, H), blk),
            pl.BlockSpec((NC, BN, H), blk3c),
            pl.BlockSpec((H, H), full),
            pl.BlockSpec((H, H), full),
            pl.BlockSpec((1, H), full),
            pl.BlockSpec((H, H), full),
            pl.BlockSpec((1, H), full),
            pl.BlockSpec((H, H), full),
            pl.BlockSpec((1, H), full),
            pl.BlockSpec((1, H), full),
            pl.BlockSpec((1, 1), full),
        ],
        out_specs=[pl.BlockSpec((BN, 1), blk)],
        out_shape=[jax.ShapeDtypeStruct((N, 1), _f32)],
        compiler_params=pltpu.CompilerParams(
            dimension_semantics=("arbitrary",)),
    )(h2, aggh, wh1a, wh1b, bh1, wh2, bh2, wout, bout, wfinr, bfin)[0]


def _sc_gather(ts, td, src, dst):
    @functools.partial(
        pl.kernel,
        out_type=(jax.ShapeDtypeStruct((E, 2, H), _f32),
                  jax.ShapeDtypeStruct((E, 2, H), _f32)),
        mesh=_mesh(),
        scratch_types=[
            pltpu.VMEM((CHUNK,), jnp.int32),
            pltpu.VMEM((CHUNK,), jnp.int32),
            pltpu.VMEM((CHUNK, 2, H), _f32),
            pltpu.VMEM((CHUNK, 2, H), _f32),
            pltpu.SemaphoreType.DMA,
            pltpu.SemaphoreType.DMA,
        ],
    )
    def k(ts_hbm, td_hbm, src_hbm, dst_hbm, gs_hbm, gd_hbm,
          idxs_v, idxd_v, gs_v, gd_v, sem1, sem2):
        cid = jax.lax.axis_index("c")
        sid = jax.lax.axis_index("s")
        wid = sid * NC + cid

        def do(chunk):
            base = chunk * CHUNK
            c1 = pltpu.async_copy(src_hbm.at[pl.ds(base, CHUNK)], idxs_v, sem1)
            c2 = pltpu.async_copy(dst_hbm.at[pl.ds(base, CHUNK)], idxd_v, sem2)
            c1.wait()
            c2.wait()
            g1 = pltpu.async_copy(ts_hbm.at[idxs_v], gs_v, sem1)
            g2 = pltpu.async_copy(td_hbm.at[idxd_v], gd_v, sem2)
            g1.wait()
            g2.wait()
            o1 = pltpu.async_copy(gs_v, gs_hbm.at[pl.ds(base, CHUNK)], sem1)
            o2 = pltpu.async_copy(gd_v, gd_hbm.at[pl.ds(base, CHUNK)], sem2)
            o1.wait()
            o2.wait()

        @pl.loop(0, FULL_ROUNDS)
        def _(g):
            do(g * NW + wid)

        @pl.when(wid < TAIL)
        def _():
            do(FULL_ROUNDS * NW + wid)

    return k(ts, td, src, dst)


def _sc_scatter(m, src, zh):
    @functools.partial(
        pl.kernel,
        out_type=jax.ShapeDtypeStruct((NC, N, H), _f32),
        mesh=_mesh(),
        scratch_types=[
            pltpu.VMEM((CHUNK,), jnp.int32),
            pltpu.VMEM((CHUNK, H), _f32),
            pltpu.VMEM_SHARED((N, H), _f32),
        ],
    )
    def k(m_hbm, src_hbm, zh_hbm, aggh_hbm, idx_v, m_v, aggh_s):
        cid = jax.lax.axis_index("c")
        sid = jax.lax.axis_index("s")
        wid = sid * NC + cid
        r0 = sid * NPT
        pltpu.sync_copy(zh_hbm.at[pl.ds(r0, NPT)], aggh_s.at[pl.ds(r0, NPT)])

        @pl.when(sid == 0)
        def _():
            pltpu.sync_copy(zh_hbm.at[pl.ds(NS * NPT, NREM)],
                            aggh_s.at[pl.ds(NS * NPT, NREM)])

        plsc.subcore_barrier()

        def do(chunk):
            base = chunk * CHUNK
            pltpu.sync_copy(src_hbm.at[pl.ds(base, CHUNK)], idx_v)
            pltpu.sync_copy(m_hbm.at[pl.ds(base, CHUNK)], m_v)
            pltpu.sync_copy(m_v, aggh_s.at[idx_v], add=True)

        @pl.loop(0, FULL_ROUNDS)
        def _(g):
            do(g * NW + wid)

        @pl.when(wid < TAIL)
        def _():
            do(FULL_ROUNDS * NW + wid)

        plsc.subcore_barrier()
        pltpu.sync_copy(aggh_s.at[pl.ds(r0, NPT)],
                        aggh_hbm.at[cid, pl.ds(r0, NPT)])

        @pl.when(sid == 0)
        def _():
            pltpu.sync_copy(aggh_s.at[pl.ds(NS * NPT, NREM)],
                            aggh_hbm.at[cid, pl.ds(NS * NPT, NREM)])

    return k(m, src, zh)


def kernel(h, x, edge_index, edge_attr, W_in, b_in, We1, be1, We2, be2,
           Wx1, bx1, Wx2, bx2, Wh1, bh1, Wh2, bh2, W_out, b_out, W_fin, b_fin):
    src = edge_index[0]
    dst = edge_index[1]
    xpad = jnp.pad(x, ((0, 0), (0, H - x.shape[1])))

    a_l = [We1[l][:H] for l in range(2)]
    b_l = [We1[l][H:2 * H] for l in range(2)]
    wr_l = [We1[l][2 * H:2 * H + 1] for l in range(2)]
    wea_l = [We1[l][2 * H + 1:] for l in range(2)]
    be1_l = [be1[l].reshape(1, H) for l in range(2)]
    be2_l = [be2[l].reshape(1, H) for l in range(2)]
    bx1_0 = bx1[0].reshape(1, H)
    wx2_0 = Wx2[0].reshape(1, H)
    bx2_0 = bx2[0].reshape(1, 1)
    wh1a_l = [Wh1[l][:H] for l in range(2)]
    wh1b_l = [Wh1[l][H:] for l in range(2)]
    bh1_l = [bh1[l].reshape(1, H) for l in range(2)]
    bh2_l = [bh2[l].reshape(1, H) for l in range(2)]
    b_in2 = b_in.reshape(1, H)
    b_out2 = b_out.reshape(1, H)
    wfin_r = W_fin.reshape(1, H)
    bfin2 = b_fin.reshape(1, 1)
    zh = jnp.zeros((N, H), _f32)

    h1, ts0, td0 = _tc_prep(h, xpad, W_in, b_in2, a_l[0], b_l[0])
    gs0, gd0 = _sc_gather(ts0, td0, src, dst)
    m0, t0 = _tc_edge0(gs0, gd0, edge_attr, wr_l[0], wea_l[0], be1_l[0],
                       We2[0], be2_l[0], Wx1[0], bx1_0, wx2_0, bx2_0)
    aggh0 = _sc_scatter(m0, src, zh)
    aggx0 = _sc_scatter(t0, src, zh)
    h2, ts1, td1 = _tc_node0(h1, xpad, aggh0, aggx0, wh1a_l[0], wh1b_l[0],
                             bh1_l[0], Wh2[0], bh2_l[0], a_l[1], b_l[1])

    gs1, gd1 = _sc_gather(ts1, td1, src, dst)
    m1 = _tc_edge1(gs1, gd1, edge_attr, wr_l[1], wea_l[1], be1_l[1],
                   We2[1], be2_l[1])
    aggh1 = _sc_scatter(m1, src, zh)
    out = _tc_node1(h2, aggh1, wh1a_l[1], wh1b_l[1], bh1_l[1], Wh2[1],
                    bh2_l[1], W_out, b_out2, wfin_r, bfin2)
    return out


# speedup vs baseline: 3.7425x; 1.7496x over previous
import functools

import jax
import jax.numpy as jnp
from jax.experimental import pallas as pl
from jax.experimental.pallas import tpu as pltpu
from jax.experimental.pallas import tpu_sc as plsc

N = 10000
E = 320000
H = 128
HH = 64
DE = 16
AVG_DEG = float(E) / float(N)

NC = 2
NS = 16
NW = NC * NS
CHUNK = 128
NCHUNK = E // CHUNK
FULL_ROUNDS = NCHUNK // NW
TAIL = NCHUNK - FULL_ROUNDS * NW

BN = 1000
BE = 1280
NPT = 624
NREM = N - NS * NPT

_f32 = jnp.float32
_bf16 = jnp.bfloat16


def _silu(v):
    return v * jax.nn.sigmoid(v)


_u32 = jnp.uint32


def _pack_bf16(p):
    ulo = jax.lax.bitcast_convert_type(p[:, :HH], _u32)
    uhi = jax.lax.bitcast_convert_type(p[:, HH:], _u32)
    rnd = _u32(0x8000)
    msk = _u32(0xFFFF0000)
    word = (((uhi + rnd) & msk) | ((ulo + rnd) >> 16))
    return jax.lax.bitcast_convert_type(word, _f32)


def _unpack_bf16(w):
    u = jax.lax.bitcast_convert_type(w, _u32)
    lo = jax.lax.bitcast_convert_type(u << 16, _f32)
    hi = jax.lax.bitcast_convert_type(u & _u32(0xFFFF0000), _f32)
    return jnp.concatenate([lo, hi], axis=1)


def _mesh():
    return plsc.VectorSubcoreMesh(core_axis_name="c", subcore_axis_name="s",
                                  num_cores=NC, num_subcores=NS)


def _prep_body(h_ref, xp_ref, win_ref, bin_ref, a_ref, b_ref,
               h1_ref, ts_ref, td_ref):
    h1 = jnp.dot(h_ref[...], win_ref[...],
                 preferred_element_type=_f32) + bin_ref[...]
    h1_ref[...] = h1
    xp = xp_ref[...]
    p = jnp.dot(h1, a_ref[...], preferred_element_type=_f32)
    q = jnp.dot(h1, b_ref[...], preferred_element_type=_f32)
    ts_ref[...] = jnp.concatenate([_pack_bf16(p), xp], axis=1)
    td_ref[...] = jnp.concatenate([_pack_bf16(q), xp], axis=1)


def _tc_prep(h, xpad, w_in, b_in, a0, b0):
    grid = (N // BN,)
    blk = lambda i: (i, 0)
    full = lambda i: (0, 0)
    return pl.pallas_call(
        _prep_body,
        grid=grid,
        in_specs=[
            pl.BlockSpec((BN, H), blk),
            pl.BlockSpec((BN, HH), blk),
            pl.BlockSpec((H, H), full),
            pl.BlockSpec((1, H), full),
            pl.BlockSpec((H, H), full),
            pl.BlockSpec((H, H), full),
        ],
        out_specs=[
            pl.BlockSpec((BN, H), blk),
            pl.BlockSpec((BN, H), blk),
            pl.BlockSpec((BN, H), blk),
        ],
        out_shape=[
            jax.ShapeDtypeStruct((N, H), _f32),
            jax.ShapeDtypeStruct((N, H), _f32),
            jax.ShapeDtypeStruct((N, H), _f32),
        ],
        compiler_params=pltpu.CompilerParams(
            dimension_semantics=("arbitrary",)),
    )(h, xpad, w_in, b_in, a0, b0)


def _edge0_body(gs_ref, gd_ref, ea_ref, wr_ref, wea_ref, be1_ref,
                we2_ref, be2_ref, wx1_ref, bx1_ref, wx2t_ref, bx2_ref,
                ones_ref, m_ref, t_ref):
    gs = gs_ref[...]
    gd = gd_ref[...]
    ps = _unpack_bf16(gs[:, :HH])
    qd = _unpack_bf16(gd[:, :HH])
    diff = gs[:, HH:] - gd[:, HH:]
    r2b = jnp.dot(diff * diff, ones_ref[...], preferred_element_type=_f32)
    a = (ps + qd + r2b * wr_ref[...] +
         jnp.dot(ea_ref[...], wea_ref[...], preferred_element_type=_f32) +
         be1_ref[...])
    m = _silu(jnp.dot(_silu(a), we2_ref[...],
                      preferred_element_type=_f32) + be2_ref[...])
    u = _silu(jnp.dot(m, wx1_ref[...],
                      preferred_element_type=_f32) + bx1_ref[...])
    wb = jnp.dot(u, wx2t_ref[...], preferred_element_type=_f32) + bx2_ref[...]
    m_ref[...] = m
    t_ref[...] = jnp.concatenate(
        [jnp.zeros((gs.shape[0], HH), _f32), diff * wb], axis=1)


def _tc_edge0(gs, gd, ea, wr, wea, be1, we2, be2, wx1, bx1, wx2t, bx2,
              ones_c):
    grid = (E // BE,)
    blk = lambda i: (i, 0)
    full = lambda i: (0, 0)
    return pl.pallas_call(
        _edge0_body,
        grid=grid,
        in_specs=[
            pl.BlockSpec((BE, H), blk),
            pl.BlockSpec((BE, H), blk),
            pl.BlockSpec((BE, DE), blk),
            pl.BlockSpec((1, H), full),
            pl.BlockSpec((DE, H), full),
            pl.BlockSpec((1, H), full),
            pl.BlockSpec((H, H), full),
            pl.BlockSpec((1, H), full),
            pl.BlockSpec((H, H), full),
            pl.BlockSpec((1, H), full),
            pl.BlockSpec((H, HH), full),
            pl.BlockSpec((1, 1), full),
            pl.BlockSpec((HH, H), full),
        ],
        out_specs=[
            pl.BlockSpec((BE, H), blk),
            pl.BlockSpec((BE, H), blk),
        ],
        out_shape=[
            jax.ShapeDtypeStruct((E, H), _f32),
            jax.ShapeDtypeStruct((E, H), _f32),
        ],
        compiler_params=pltpu.CompilerParams(
            dimension_semantics=("arbitrary",)),
    )(gs, gd, ea, wr, wea, be1, we2, be2, wx1, bx1, wx2t, bx2, ones_c)


def _edge1_body(gs_ref, gd_ref, ea_ref, wr_ref, wea_ref, be1_ref,
                we2_ref, be2_ref, ones_ref, m_ref):
    gs = gs_ref[...]
    gd = gd_ref[...]
    ps = _unpack_bf16(gs[:, :HH])
    qd = _unpack_bf16(gd[:, :HH])
    diff = gs[:, HH:] - gd[:, HH:]
    r2b = jnp.dot(diff * diff, ones_ref[...], preferred_element_type=_f32)
    a = (ps + qd + r2b * wr_ref[...] +
         jnp.dot(ea_ref[...], wea_ref[...], preferred_element_type=_f32) +
         be1_ref[...])
    m_ref[...] = _silu(jnp.dot(_silu(a), we2_ref[...],
                               preferred_element_type=_f32) + be2_ref[...])


def _tc_edge1(gs, gd, ea, wr, wea, be1, we2, be2, ones_c):
    grid = (E // BE,)
    blk = lambda i: (i, 0)
    full = lambda i: (0, 0)
    return pl.pallas_call(
        _edge1_body,
        grid=grid,
        in_specs=[
            pl.BlockSpec((BE, H), blk),
            pl.BlockSpec((BE, H), blk),
            pl.BlockSpec((BE, DE), blk),
            pl.BlockSpec((1, H), full),
            pl.BlockSpec((DE, H), full),
            pl.BlockSpec((1, H), full),
            pl.BlockSpec((H, H), full),
            pl.BlockSpec((1, H), full),
            pl.BlockSpec((HH, H), full),
        ],
        out_specs=[pl.BlockSpec((BE, H), blk)],
        out_shape=[jax.ShapeDtypeStruct((E, H), _f32)],
        compiler_params=pltpu.CompilerParams(
            dimension_semantics=("arbitrary",)),
    )(gs, gd, ea, wr, wea, be1, we2, be2, ones_c)[0]


def _node0_body(h1_ref, xp_ref, aggh_ref, aggx_ref, wh1a_ref, wh1b_ref,
                bh1_ref, wh2_ref, bh2_ref, a1_ref, b1_ref,
                h2_ref, ts_ref, td_ref):
    h1 = h1_ref[...]
    aggh = aggh_ref[0] + aggh_ref[1]
    pre = (jnp.dot(h1, wh1a_ref[...], preferred_element_type=_f32) +
           jnp.dot(aggh, wh1b_ref[...], preferred_element_type=_f32) +
           bh1_ref[...])
    upd = jnp.dot(_silu(pre), wh2_ref[...],
                  preferred_element_type=_f32) + bh2_ref[...]
    h2 = h1 + upd
    h2_ref[...] = h2
    aggx_arr = aggx_ref[...]
    aggx = aggx_arr[0, :, HH:] + aggx_arr[1, :, HH:]
    xn = xp_ref[...] + aggx * (1.0 / AVG_DEG)
    p = jnp.dot(h2, a1_ref[...], preferred_element_type=_f32)
    q = jnp.dot(h2, b1_ref[...], preferred_element_type=_f32)
    ts_ref[...] = jnp.concatenate([_pack_bf16(p), xn], axis=1)
    td_ref[...] = jnp.concatenate([_pack_bf16(q), xn], axis=1)


def _tc_node0(h1, xpad, aggh, aggx, wh1a, wh1b, bh1, wh2, bh2, a1, b1):
    grid = (N // BN,)
    blk = lambda i: (i, 0)
    blk3c = lambda i: (0, i, 0)
    full = lambda i: (0, 0)
    return pl.pallas_call(
        _node0_body,
        grid=grid,
        in_specs=[
            pl.BlockSpec((BN, H), blk),
            pl.BlockSpec((BN, HH), blk),
            pl.BlockSpec((NC, BN, H), blk3c),
            pl.BlockSpec((NC, BN, H), blk3c),
            pl.BlockSpec((H, H), full),
            pl.BlockSpec((H, H), full),
            pl.BlockSpec((1, H), full),
            pl.BlockSpec((H, H), full),
            pl.BlockSpec((1, H), full),
            pl.BlockSpec((H, H), full),
            pl.BlockSpec((H, H), full),
        ],
        out_specs=[
            pl.BlockSpec((BN, H), blk),
            pl.BlockSpec((BN, H), blk),
            pl.BlockSpec((BN, H), blk),
        ],
        out_shape=[
            jax.ShapeDtypeStruct((N, H), _f32),
            jax.ShapeDtypeStruct((N, H), _f32),
            jax.ShapeDtypeStruct((N, H), _f32),
        ],
        compiler_params=pltpu.CompilerParams(
            dimension_semantics=("arbitrary",)),
    )(h1, xpad, aggh, aggx, wh1a, wh1b, bh1, wh2, bh2, a1, b1)


def _node1_body(h2_ref, aggh_ref, wh1a_ref, wh1b_ref, bh1_ref, wh2_ref,
                bh2_ref, wout_ref, bout_ref, wfin_ref, bfin_ref, out_ref):
    h2 = h2_ref[...]
    aggh = aggh_ref[0] + aggh_ref[1]
    pre = (jnp.dot(h2, wh1a_ref[...], preferred_element_type=_f32) +
           jnp.dot(aggh, wh1b_ref[...], preferred_element_type=_f32) +
           bh1_ref[...])
    upd = jnp.dot(_silu(pre), wh2_ref[...],
                  preferred_element_type=_f32) + bh2_ref[...]
    hf = h2 + upd
    ho = jnp.dot(hf, wout_ref[...], preferred_element_type=_f32) + bout_ref[...]
    out_ref[...] = jnp.sum(ho * wfin_ref[...], axis=1,
                           keepdims=True) + bfin_ref[...]


def _tc_node1(h2, aggh, wh1a, wh1b, bh1, wh2, bh2, wout, bout, wfinr, bfin):
    grid = (N // BN,)
    blk = lambda i: (i, 0)
    blk3c = lambda i: (0, i, 0)
    full = lambda i: (0, 0)
    return pl.pallas_call(
        _node1_body,
        grid=grid,
        in_specs=[
            pl.BlockSpec((BN, H), blk),
            pl.BlockSpec((NC, BN, H), blk3c),
            pl.BlockSpec((H, H), full),
            pl.BlockSpec((H, H), full),
            pl.BlockSpec((1, H), full),
            pl.BlockSpec((H, H), full),
            pl.BlockSpec((1, H), full),
            pl.BlockSpec((H, H), full),
            pl.BlockSpec((1, H), full),
            pl.BlockSpec((1, H), full),
            pl.BlockSpec((1, 1), full),
        ],
        out_specs=[pl.BlockSpec((BN, 1), blk)],
        out_shape=[jax.ShapeDtypeStruct((N, 1), _f32)],
        compiler_params=pltpu.CompilerParams(
            dimension_semantics=("arbitrary",)),
    )(h2, aggh, wh1a, wh1b, bh1, wh2, bh2, wout, bout, wfinr, bfin)[0]


def _sc_gather(ts, td, src, dst):
    @functools.partial(
        pl.kernel,
        out_type=(jax.ShapeDtypeStruct((E, H), _f32),
                  jax.ShapeDtypeStruct((E, H), _f32)),
        mesh=_mesh(),
        scratch_types=[
            pltpu.VMEM((CHUNK,), jnp.int32),
            pltpu.VMEM((CHUNK,), jnp.int32),
            pltpu.VMEM((CHUNK, H), _f32),
            pltpu.VMEM((CHUNK, H), _f32),
            pltpu.SemaphoreType.DMA,
            pltpu.SemaphoreType.DMA,
        ],
    )
    def k(ts_hbm, td_hbm, src_hbm, dst_hbm, gs_hbm, gd_hbm,
          idxs_v, idxd_v, gs_v, gd_v, sem1, sem2):
        cid = jax.lax.axis_index("c")
        sid = jax.lax.axis_index("s")
        wid = sid * NC + cid

        def do(chunk):
            base = chunk * CHUNK
            c1 = pltpu.async_copy(src_hbm.at[pl.ds(base, CHUNK)], idxs_v, sem1)
            c2 = pltpu.async_copy(dst_hbm.at[pl.ds(base, CHUNK)], idxd_v, sem2)
            c1.wait()
            c2.wait()
            g1 = pltpu.async_copy(ts_hbm.at[idxs_v], gs_v, sem1)
            g2 = pltpu.async_copy(td_hbm.at[idxd_v], gd_v, sem2)
            g1.wait()
            g2.wait()
            o1 = pltpu.async_copy(gs_v, gs_hbm.at[pl.ds(base, CHUNK)], sem1)
            o2 = pltpu.async_copy(gd_v, gd_hbm.at[pl.ds(base, CHUNK)], sem2)
            o1.wait()
            o2.wait()

        @pl.loop(0, FULL_ROUNDS)
        def _(g):
            do(g * NW + wid)

        @pl.when(wid < TAIL)
        def _():
            do(FULL_ROUNDS * NW + wid)

    return k(ts, td, src, dst)


def _sc_scatter(m, src, zh):
    @functools.partial(
        pl.kernel,
        out_type=jax.ShapeDtypeStruct((NC, N, H), _f32),
        mesh=_mesh(),
        scratch_types=[
            pltpu.VMEM((CHUNK,), jnp.int32),
            pltpu.VMEM((CHUNK, H), _f32),
            pltpu.VMEM_SHARED((N, H), _f32),
        ],
    )
    def k(m_hbm, src_hbm, zh_hbm, aggh_hbm, idx_v, m_v, aggh_s):
        cid = jax.lax.axis_index("c")
        sid = jax.lax.axis_index("s")
        wid = sid * NC + cid
        r0 = sid * NPT
        pltpu.sync_copy(zh_hbm.at[pl.ds(r0, NPT)], aggh_s.at[pl.ds(r0, NPT)])

        @pl.when(sid == 0)
        def _():
            pltpu.sync_copy(zh_hbm.at[pl.ds(NS * NPT, NREM)],
                            aggh_s.at[pl.ds(NS * NPT, NREM)])

        plsc.subcore_barrier()

        def do(chunk):
            base = chunk * CHUNK
            pltpu.sync_copy(src_hbm.at[pl.ds(base, CHUNK)], idx_v)
            pltpu.sync_copy(m_hbm.at[pl.ds(base, CHUNK)], m_v)
            pltpu.sync_copy(m_v, aggh_s.at[idx_v], add=True)

        @pl.loop(0, FULL_ROUNDS)
        def _(g):
            do(g * NW + wid)

        @pl.when(wid < TAIL)
        def _():
            do(FULL_ROUNDS * NW + wid)

        plsc.subcore_barrier()
        pltpu.sync_copy(aggh_s.at[pl.ds(r0, NPT)],
                        aggh_hbm.at[cid, pl.ds(r0, NPT)])

        @pl.when(sid == 0)
        def _():
            pltpu.sync_copy(aggh_s.at[pl.ds(NS * NPT, NREM)],
                            aggh_hbm.at[cid, pl.ds(NS * NPT, NREM)])

    return k(m, src, zh)


def kernel(h, x, edge_index, edge_attr, W_in, b_in, We1, be1, We2, be2,
           Wx1, bx1, Wx2, bx2, Wh1, bh1, Wh2, bh2, W_out, b_out, W_fin, b_fin):
    src = edge_index[0]
    dst = edge_index[1]
    xpad = jnp.pad(x, ((0, 0), (0, HH - x.shape[1])))

    a_l = [We1[l][:H] for l in range(2)]
    b_l = [We1[l][H:2 * H] for l in range(2)]
    wr_l = [We1[l][2 * H:2 * H + 1] for l in range(2)]
    wea_l = [We1[l][2 * H + 1:] for l in range(2)]
    be1_l = [be1[l].reshape(1, H) for l in range(2)]
    be2_l = [be2[l].reshape(1, H) for l in range(2)]
    bx1_0 = bx1[0].reshape(1, H)
    wx2t_0 = jnp.broadcast_to(Wx2[0], (H, HH))
    bx2_0 = bx2[0].reshape(1, 1)
    wh1a_l = [Wh1[l][:H] for l in range(2)]
    wh1b_l = [Wh1[l][H:] for l in range(2)]
    bh1_l = [bh1[l].reshape(1, H) for l in range(2)]
    bh2_l = [bh2[l].reshape(1, H) for l in range(2)]
    b_in2 = b_in.reshape(1, H)
    b_out2 = b_out.reshape(1, H)
    wfin_r = W_fin.reshape(1, H)
    bfin2 = b_fin.reshape(1, 1)
    zh = jnp.zeros((N, H), _f32)
    ones_c = jnp.ones((HH, H), _f32)

    h1, ts0, td0 = _tc_prep(h, xpad, W_in, b_in2, a_l[0], b_l[0])
    gs0, gd0 = _sc_gather(ts0, td0, src, dst)
    m0, t0 = _tc_edge0(gs0, gd0, edge_attr, wr_l[0], wea_l[0], be1_l[0],
                       We2[0], be2_l[0], Wx1[0], bx1_0, wx2t_0, bx2_0, ones_c)
    aggh0 = _sc_scatter(m0, src, zh)
    aggx0 = _sc_scatter(t0, src, zh)
    h2, ts1, td1 = _tc_node0(h1, xpad, aggh0, aggx0, wh1a_l[0], wh1b_l[0],
                             bh1_l[0], Wh2[0], bh2_l[0], a_l[1], b_l[1])

    gs1, gd1 = _sc_gather(ts1, td1, src, dst)
    m1 = _tc_edge1(gs1, gd1, edge_attr, wr_l[1], wea_l[1], be1_l[1],
                   We2[1], be2_l[1], ones_c)
    aggh1 = _sc_scatter(m1, src, zh)
    out = _tc_node1(h2, aggh1, wh1a_l[1], wh1b_l[1], bh1_l[1], Wh2[1],
                    bh2_l[1], W_out, b_out2, wfin_r, bfin2)
    return out


# speedup vs baseline: 4.0502x; 1.0822x over previous
import functools

import jax
import jax.numpy as jnp
from jax.experimental import pallas as pl
from jax.experimental.pallas import tpu as pltpu
from jax.experimental.pallas import tpu_sc as plsc

N = 10000
E = 320000
H = 128
HH = 64
DE = 16
AVG_DEG = float(E) / float(N)

NC = 2
NS = 16
NW = NC * NS
CHUNK = 128
NCHUNK = E // CHUNK
RPT = 80
RPT_LAST = NCHUNK - (NW - 1) * RPT

BN = 1000
BE = 1280
NPT = 624
NREM = N - NS * NPT

_f32 = jnp.float32
_bf16 = jnp.bfloat16


def _silu(v):
    return v * jax.nn.sigmoid(v)


_u32 = jnp.uint32


def _pack_bf16(p):
    ulo = jax.lax.bitcast_convert_type(p[:, :HH], _u32)
    uhi = jax.lax.bitcast_convert_type(p[:, HH:], _u32)
    rnd = _u32(0x8000)
    msk = _u32(0xFFFF0000)
    word = (((uhi + rnd) & msk) | ((ulo + rnd) >> 16))
    return jax.lax.bitcast_convert_type(word, _f32)


def _unpack_bf16(w):
    u = jax.lax.bitcast_convert_type(w, _u32)
    lo = jax.lax.bitcast_convert_type(u << 16, _f32)
    hi = jax.lax.bitcast_convert_type(u & _u32(0xFFFF0000), _f32)
    return jnp.concatenate([lo, hi], axis=1)


def _mesh():
    return plsc.VectorSubcoreMesh(core_axis_name="c", subcore_axis_name="s",
                                  num_cores=NC, num_subcores=NS)


def _prep_body(h_ref, xp_ref, win_ref, bin_ref, a_ref, b_ref,
               h1_ref, ts_ref, td_ref):
    h1 = jnp.dot(h_ref[...], win_ref[...],
                 preferred_element_type=_f32) + bin_ref[...]
    h1_ref[...] = h1
    xp = xp_ref[...]
    p = jnp.dot(h1, a_ref[...], preferred_element_type=_f32)
    q = jnp.dot(h1, b_ref[...], preferred_element_type=_f32)
    ts_ref[...] = jnp.concatenate([_pack_bf16(p), xp], axis=1)
    td_ref[...] = jnp.concatenate([_pack_bf16(q), xp], axis=1)


def _tc_prep(h, xpad, w_in, b_in, a0, b0):
    grid = (N // BN,)
    blk = lambda i: (i, 0)
    full = lambda i: (0, 0)
    return pl.pallas_call(
        _prep_body,
        grid=grid,
        in_specs=[
            pl.BlockSpec((BN, H), blk),
            pl.BlockSpec((BN, HH), blk),
            pl.BlockSpec((H, H), full),
            pl.BlockSpec((1, H), full),
            pl.BlockSpec((H, H), full),
            pl.BlockSpec((H, H), full),
        ],
        out_specs=[
            pl.BlockSpec((BN, H), blk),
            pl.BlockSpec((BN, H), blk),
            pl.BlockSpec((BN, H), blk),
        ],
        out_shape=[
            jax.ShapeDtypeStruct((N, H), _f32),
            jax.ShapeDtypeStruct((N, H), _f32),
            jax.ShapeDtypeStruct((N, H), _f32),
        ],
        compiler_params=pltpu.CompilerParams(
            dimension_semantics=("arbitrary",)),
    )(h, xpad, w_in, b_in, a0, b0)


def _edge0_body(gs_ref, gd_ref, ea_ref, wr_ref, wea_ref, be1_ref,
                we2_ref, be2_ref, wx1_ref, bx1_ref, wx2t_ref, bx2_ref,
                ones_ref, m_ref, t_ref):
    gs = gs_ref[...]
    gd = gd_ref[...]
    ps = _unpack_bf16(gs[:, :HH])
    qd = _unpack_bf16(gd[:, :HH])
    diff = gs[:, HH:] - gd[:, HH:]
    r2b = jnp.dot(diff * diff, ones_ref[...], preferred_element_type=_f32,
                  precision=jax.lax.Precision.HIGHEST)
    a = (ps + qd + r2b * wr_ref[...] +
         jnp.dot(ea_ref[...], wea_ref[...], preferred_element_type=_f32) +
         be1_ref[...])
    m = _silu(jnp.dot(_silu(a), we2_ref[...],
                      preferred_element_type=_f32) + be2_ref[...])
    u = _silu(jnp.dot(m, wx1_ref[...],
                      preferred_element_type=_f32) + bx1_ref[...])
    wb = jnp.dot(u, wx2t_ref[...], preferred_element_type=_f32) + bx2_ref[...]
    m_ref[...] = m
    t_ref[...] = jnp.concatenate(
        [jnp.zeros((gs.shape[0], HH), _f32), diff * wb], axis=1)


def _tc_edge0(gs, gd, ea, wr, wea, be1, we2, be2, wx1, bx1, wx2t, bx2,
              ones_c):
    grid = (E // BE,)
    blk = lambda i: (i, 0)
    full = lambda i: (0, 0)
    return pl.pallas_call(
        _edge0_body,
        grid=grid,
        in_specs=[
            pl.BlockSpec((BE, H), blk),
            pl.BlockSpec((BE, H), blk),
            pl.BlockSpec((BE, DE), blk),
            pl.BlockSpec((1, H), full),
            pl.BlockSpec((DE, H), full),
            pl.BlockSpec((1, H), full),
            pl.BlockSpec((H, H), full),
            pl.BlockSpec((1, H), full),
            pl.BlockSpec((H, H), full),
            pl.BlockSpec((1, H), full),
            pl.BlockSpec((H, HH), full),
            pl.BlockSpec((1, 1), full),
            pl.BlockSpec((HH, H), full),
        ],
        out_specs=[
            pl.BlockSpec((BE, H), blk),
            pl.BlockSpec((BE, H), blk),
        ],
        out_shape=[
            jax.ShapeDtypeStruct((E, H), _f32),
            jax.ShapeDtypeStruct((E, H), _f32),
        ],
        compiler_params=pltpu.CompilerParams(
            dimension_semantics=("arbitrary",)),
    )(gs, gd, ea, wr, wea, be1, we2, be2, wx1, bx1, wx2t, bx2, ones_c)


def _edge1_body(gs_ref, gd_ref, ea_ref, wr_ref, wea_ref, be1_ref,
                we2_ref, be2_ref, ones_ref, m_ref):
    gs = gs_ref[...]
    gd = gd_ref[...]
    ps = _unpack_bf16(gs[:, :HH])
    qd = _unpack_bf16(gd[:, :HH])
    diff = gs[:, HH:] - gd[:, HH:]
    r2b = jnp.dot(diff * diff, ones_ref[...], preferred_element_type=_f32,
                  precision=jax.lax.Precision.HIGHEST)
    a = (ps + qd + r2b * wr_ref[...] +
         jnp.dot(ea_ref[...], wea_ref[...], preferred_element_type=_f32) +
         be1_ref[...])
    m_ref[...] = _silu(jnp.dot(_silu(a), we2_ref[...],
                               preferred_element_type=_f32) + be2_ref[...])


def _tc_edge1(gs, gd, ea, wr, wea, be1, we2, be2, ones_c):
    grid = (E // BE,)
    blk = lambda i: (i, 0)
    full = lambda i: (0, 0)
    return pl.pallas_call(
        _edge1_body,
        grid=grid,
        in_specs=[
            pl.BlockSpec((BE, H), blk),
            pl.BlockSpec((BE, H), blk),
            pl.BlockSpec((BE, DE), blk),
            pl.BlockSpec((1, H), full),
            pl.BlockSpec((DE, H), full),
            pl.BlockSpec((1, H), full),
            pl.BlockSpec((H, H), full),
            pl.BlockSpec((1, H), full),
            pl.BlockSpec((HH, H), full),
        ],
        out_specs=[pl.BlockSpec((BE, H), blk)],
        out_shape=[jax.ShapeDtypeStruct((E, H), _f32)],
        compiler_params=pltpu.CompilerParams(
            dimension_semantics=("arbitrary",)),
    )(gs, gd, ea, wr, wea, be1, we2, be2, ones_c)[0]


def _node0_body(h1_ref, xp_ref, aggh_ref, aggx_ref, wh1a_ref, wh1b_ref,
                bh1_ref, wh2_ref, bh2_ref, a1_ref, b1_ref,
                h2_ref, ts_ref, td_ref):
    h1 = h1_ref[...]
    aggh = aggh_ref[0] + aggh_ref[1]
    pre = (jnp.dot(h1, wh1a_ref[...], preferred_element_type=_f32) +
           jnp.dot(aggh, wh1b_ref[...], preferred_element_type=_f32) +
           bh1_ref[...])
    upd = jnp.dot(_silu(pre), wh2_ref[...],
                  preferred_element_type=_f32) + bh2_ref[...]
    h2 = h1 + upd
    h2_ref[...] = h2
    aggx_arr = aggx_ref[...]
    aggx = aggx_arr[0, :, HH:] + aggx_arr[1, :, HH:]
    xn = xp_ref[...] + aggx * (1.0 / AVG_DEG)
    p = jnp.dot(h2, a1_ref[...], preferred_element_type=_f32)
    q = jnp.dot(h2, b1_ref[...], preferred_element_type=_f32)
    ts_ref[...] = jnp.concatenate([_pack_bf16(p), xn], axis=1)
    td_ref[...] = jnp.concatenate([_pack_bf16(q), xn], axis=1)


def _tc_node0(h1, xpad, aggh, aggx, wh1a, wh1b, bh1, wh2, bh2, a1, b1):
    grid = (N // BN,)
    blk = lambda i: (i, 0)
    blk3c = lambda i: (0, i, 0)
    full = lambda i: (0, 0)
    return pl.pallas_call(
        _node0_body,
        grid=grid,
        in_specs=[
            pl.BlockSpec((BN, H), blk),
            pl.BlockSpec((BN, HH), blk),
            pl.BlockSpec((NC, BN, H), blk3c),
            pl.BlockSpec((NC, BN, H), blk3c),
            pl.BlockSpec((H, H), full),
            pl.BlockSpec((H, H), full),
            pl.BlockSpec((1, H), full),
            pl.BlockSpec((H, H), full),
            pl.BlockSpec((1, H), full),
            pl.BlockSpec((H, H), full),
            pl.BlockSpec((H, H), full),
        ],
        out_specs=[
            pl.BlockSpec((BN, H), blk),
            pl.BlockSpec((BN, H), blk),
            pl.BlockSpec((BN, H), blk),
        ],
        out_shape=[
            jax.ShapeDtypeStruct((N, H), _f32),
            jax.ShapeDtypeStruct((N, H), _f32),
            jax.ShapeDtypeStruct((N, H), _f32),
        ],
        compiler_params=pltpu.CompilerParams(
            dimension_semantics=("arbitrary",)),
    )(h1, xpad, aggh, aggx, wh1a, wh1b, bh1, wh2, bh2, a1, b1)


def _node1_body(h2_ref, aggh_ref, wh1a_ref, wh1b_ref, bh1_ref, wh2_ref,
                bh2_ref, wout_ref, bout_ref, wfin_ref, bfin_ref, out_ref):
    h2 = h2_ref[...]
    aggh = aggh_ref[0] + aggh_ref[1]
    pre = (jnp.dot(h2, wh1a_ref[...], preferred_element_type=_f32) +
           jnp.dot(aggh, wh1b_ref[...], preferred_element_type=_f32) +
           bh1_ref[...])
    upd = jnp.dot(_silu(pre), wh2_ref[...],
                  preferred_element_type=_f32) + bh2_ref[...]
    hf = h2 + upd
    ho = jnp.dot(hf, wout_ref[...], preferred_element_type=_f32) + bout_ref[...]
    out_ref[...] = jnp.dot(ho, wfin_ref[...],
                           preferred_element_type=_f32) + bfin_ref[...]


def _tc_node1(h2, aggh, wh1a, wh1b, bh1, wh2, bh2, wout, bout, wfinr, bfin):
    grid = (N // BN,)
    blk = lambda i: (i, 0)
    blk3c = lambda i: (0, i, 0)
    full = lambda i: (0, 0)
    return pl.pallas_call(
        _node1_body,
        grid=grid,
        in_specs=[
            pl.BlockSpec((BN, H), blk),
            pl.BlockSpec((NC, BN, H), blk3c),
            pl.BlockSpec((H, H), full),
            pl.BlockSpec((H, H), full),
            pl.BlockSpec((1, H), full),
            pl.BlockSpec((H, H), full),
            pl.BlockSpec((1, H), full),
            pl.BlockSpec((H, H), full),
            pl.BlockSpec((1, H), full),
            pl.BlockSpec((H, 1), full),
            pl.BlockSpec((1, 1), full),
        ],
        out_specs=[pl.BlockSpec((BN, 1), blk)],
        out_shape=[jax.ShapeDtypeStruct((N, 1), _f32)],
        compiler_params=pltpu.CompilerParams(
            dimension_semantics=("arbitrary",)),
    )(h2, aggh, wh1a, wh1b, bh1, wh2, bh2, wout, bout, wfinr, bfin)[0]


def _sc_gather(ts, td, src2d, dst2d):
    @functools.partial(
        pl.kernel,
        out_type=(jax.ShapeDtypeStruct((E, H), _f32),
                  jax.ShapeDtypeStruct((E, H), _f32)),
        mesh=_mesh(),
        scratch_types=[
            pltpu.VMEM((RPT, 1, CHUNK), jnp.int32),
            pltpu.VMEM((RPT, 1, CHUNK), jnp.int32),
            pltpu.VMEM((CHUNK, H), _f32),
            pltpu.VMEM((CHUNK, H), _f32),
            pltpu.VMEM((CHUNK, H), _f32),
            pltpu.VMEM((CHUNK, H), _f32),
            pltpu.SemaphoreType.DMA,
            pltpu.SemaphoreType.DMA,
            pltpu.SemaphoreType.DMA,
            pltpu.SemaphoreType.DMA,
            pltpu.SemaphoreType.DMA,
            pltpu.SemaphoreType.DMA,
            pltpu.SemaphoreType.DMA,
            pltpu.SemaphoreType.DMA,
        ],
    )
    def k(ts_hbm, td_hbm, src_hbm, dst_hbm, gs_hbm, gd_hbm,
          idxs_v, idxd_v, gs0, gd0, gs1, gd1,
          sgs0, sgd0, sgs1, sgd1, sws0, swd0, sws1, swd1):
        cid = jax.lax.axis_index("c")
        sid = jax.lax.axis_index("s")
        wid = sid * NC + cid
        row0 = wid * RPT
        npairs = jnp.where(wid == NW - 1, RPT_LAST // 2, RPT // 2)

        @pl.when(wid < NW - 1)
        def _():
            c1 = pltpu.async_copy(src_hbm.at[pl.ds(row0, RPT)], idxs_v, sgs0)
            c2 = pltpu.async_copy(dst_hbm.at[pl.ds(row0, RPT)], idxd_v, sgd0)
            c1.wait()
            c2.wait()

        @pl.when(wid == NW - 1)
        def _():
            c1 = pltpu.async_copy(src_hbm.at[pl.ds(row0, RPT_LAST)],
                                  idxs_v.at[pl.ds(0, RPT_LAST)], sgs0)
            c2 = pltpu.async_copy(dst_hbm.at[pl.ds(row0, RPT_LAST)],
                                  idxd_v.at[pl.ds(0, RPT_LAST)], sgd0)
            c1.wait()
            c2.wait()

        @pl.loop(0, npairs)
        def _(t):
            ca = 2 * t
            cb = 2 * t + 1
            ga_s = pltpu.async_copy(ts_hbm.at[idxs_v.at[ca, 0]], gs0, sgs0)
            ga_d = pltpu.async_copy(td_hbm.at[idxd_v.at[ca, 0]], gd0, sgd0)
            gb_s = pltpu.async_copy(ts_hbm.at[idxs_v.at[cb, 0]], gs1, sgs1)
            gb_d = pltpu.async_copy(td_hbm.at[idxd_v.at[cb, 0]], gd1, sgd1)
            ga_s.wait()
            wa_s = pltpu.async_copy(
                gs0, gs_hbm.at[pl.ds((row0 + ca) * CHUNK, CHUNK)], sws0)
            ga_d.wait()
            wa_d = pltpu.async_copy(
                gd0, gd_hbm.at[pl.ds((row0 + ca) * CHUNK, CHUNK)], swd0)
            gb_s.wait()
            wb_s = pltpu.async_copy(
                gs1, gs_hbm.at[pl.ds((row0 + cb) * CHUNK, CHUNK)], sws1)
            gb_d.wait()
            wb_d = pltpu.async_copy(
                gd1, gd_hbm.at[pl.ds((row0 + cb) * CHUNK, CHUNK)], swd1)
            wa_s.wait()
            wa_d.wait()
            wb_s.wait()
            wb_d.wait()

    return k(ts, td, src2d, dst2d)


def _sc_scatter(m, src, zh):
    @functools.partial(
        pl.kernel,
        out_type=jax.ShapeDtypeStruct((NC, N, H), _f32),
        mesh=_mesh(),
        scratch_types=[
            pltpu.VMEM((RPT, 1, CHUNK), jnp.int32),
            pltpu.VMEM((CHUNK, H), _f32),
            pltpu.VMEM((CHUNK, H), _f32),
            pltpu.VMEM_SHARED((N, H), _f32),
            pltpu.SemaphoreType.DMA,
            pltpu.SemaphoreType.DMA,
            pltpu.SemaphoreType.DMA,
            pltpu.SemaphoreType.DMA,
        ],
    )
    def k(m_hbm, src_hbm, zh_hbm, aggh_hbm, idx_v, mb0, mb1, aggh_s,
          sl0, sl1, ss0, ss1):
        cid = jax.lax.axis_index("c")
        sid = jax.lax.axis_index("s")
        wid = sid * NC + cid
        row0 = wid * RPT
        npairs = jnp.where(wid == NW - 1, RPT_LAST // 2, RPT // 2)

        @pl.when(wid < NW - 1)
        def _():
            pltpu.async_copy(src_hbm.at[pl.ds(row0, RPT)], idx_v, sl0).wait()

        @pl.when(wid == NW - 1)
        def _():
            pltpu.async_copy(src_hbm.at[pl.ds(row0, RPT_LAST)],
                             idx_v.at[pl.ds(0, RPT_LAST)], sl0).wait()

        r0 = sid * NPT
        pltpu.sync_copy(zh_hbm.at[pl.ds(r0, NPT)], aggh_s.at[pl.ds(r0, NPT)])

        @pl.when(sid == 0)
        def _():
            pltpu.sync_copy(zh_hbm.at[pl.ds(NS * NPT, NREM)],
                            aggh_s.at[pl.ds(NS * NPT, NREM)])

        plsc.subcore_barrier()

        @pl.loop(0, npairs)
        def _(t):
            ca = 2 * t
            cb = 2 * t + 1
            la = pltpu.async_copy(
                m_hbm.at[pl.ds((row0 + ca) * CHUNK, CHUNK)], mb0, sl0)
            lb = pltpu.async_copy(
                m_hbm.at[pl.ds((row0 + cb) * CHUNK, CHUNK)], mb1, sl1)
            la.wait()
            sa = pltpu.async_copy(mb0, aggh_s.at[idx_v.at[ca, 0]], ss0, add=True)
            lb.wait()
            sb = pltpu.async_copy(mb1, aggh_s.at[idx_v.at[cb, 0]], ss1, add=True)
            sa.wait()
            sb.wait()

        plsc.subcore_barrier()
        pltpu.sync_copy(aggh_s.at[pl.ds(r0, NPT)],
                        aggh_hbm.at[cid, pl.ds(r0, NPT)])

        @pl.when(sid == 0)
        def _():
            pltpu.sync_copy(aggh_s.at[pl.ds(NS * NPT, NREM)],
                            aggh_hbm.at[cid, pl.ds(NS * NPT, NREM)])

    return k(m, src, zh)


def kernel(h, x, edge_index, edge_attr, W_in, b_in, We1, be1, We2, be2,
           Wx1, bx1, Wx2, bx2, Wh1, bh1, Wh2, bh2, W_out, b_out, W_fin, b_fin):
    src = edge_index[0].reshape(NCHUNK, 1, CHUNK)
    dst = edge_index[1].reshape(NCHUNK, 1, CHUNK)
    xpad = jnp.pad(x, ((0, 0), (0, HH - x.shape[1])))

    a_l = [We1[l][:H] for l in range(2)]
    b_l = [We1[l][H:2 * H] for l in range(2)]
    wr_l = [We1[l][2 * H:2 * H + 1] for l in range(2)]
    wea_l = [We1[l][2 * H + 1:] for l in range(2)]
    be1_l = [be1[l].reshape(1, H) for l in range(2)]
    be2_l = [be2[l].reshape(1, H) for l in range(2)]
    bx1_0 = bx1[0].reshape(1, H)
    wx2t_0 = jnp.broadcast_to(Wx2[0], (H, HH))
    bx2_0 = bx2[0].reshape(1, 1)
    wh1a_l = [Wh1[l][:H] for l in range(2)]
    wh1b_l = [Wh1[l][H:] for l in range(2)]
    bh1_l = [bh1[l].reshape(1, H) for l in range(2)]
    bh2_l = [bh2[l].reshape(1, H) for l in range(2)]
    b_in2 = b_in.reshape(1, H)
    b_out2 = b_out.reshape(1, H)
    wfin_r = W_fin
    bfin2 = b_fin.reshape(1, 1)
    zh = jnp.zeros((N, H), _f32)
    ones_c = jnp.ones((HH, H), _f32)

    h1, ts0, td0 = _tc_prep(h, xpad, W_in, b_in2, a_l[0], b_l[0])
    gs0, gd0 = _sc_gather(ts0, td0, src, dst)
    m0, t0 = _tc_edge0(gs0, gd0, edge_attr, wr_l[0], wea_l[0], be1_l[0],
                       We2[0], be2_l[0], Wx1[0], bx1_0, wx2t_0, bx2_0, ones_c)
    aggh0 = _sc_scatter(m0, src, zh)
    aggx0 = _sc_scatter(t0, src, zh)
    h2, ts1, td1 = _tc_node0(h1, xpad, aggh0, aggx0, wh1a_l[0], wh1b_l[0],
                             bh1_l[0], Wh2[0], bh2_l[0], a_l[1], b_l[1])

    gs1, gd1 = _sc_gather(ts1, td1, src, dst)
    m1 = _tc_edge1(gs1, gd1, edge_attr, wr_l[1], wea_l[1], be1_l[1],
                   We2[1], be2_l[1], ones_c)
    aggh1 = _sc_scatter(m1, src, zh)
    out = _tc_node1(h2, aggh1, wh1a_l[1], wh1b_l[1], bh1_l[1], Wh2[1],
                    bh2_l[1], W_out, b_out2, wfin_r, bfin2)
    return out


# speedup vs baseline: 4.6253x; 1.1420x over previous
import functools

import jax
import jax.numpy as jnp
from jax.experimental import pallas as pl
from jax.experimental.pallas import tpu as pltpu
from jax.experimental.pallas import tpu_sc as plsc

N = 10000
E = 320000
H = 128
HH = 64
DE = 16
AVG_DEG = float(E) / float(N)

NC = 2
NS = 16
NW = NC * NS
CHUNK = 128
NCHUNK = E // CHUNK
EH = E // 2
NROWS_H = NCHUNK // 2
RPT = 40
RPT_LAST = NROWS_H - (NW - 1) * RPT

BN = 1000
BE = 1280
NPT = 624
NREM = N - NS * NPT

_f32 = jnp.float32
_bf16 = jnp.bfloat16


def _silu(v):
    return v * jax.nn.sigmoid(v)


_u32 = jnp.uint32


def _pack_bf16(p):
    ulo = jax.lax.bitcast_convert_type(p[:, :HH], _u32)
    uhi = jax.lax.bitcast_convert_type(p[:, HH:], _u32)
    rnd = _u32(0x8000)
    msk = _u32(0xFFFF0000)
    word = (((uhi + rnd) & msk) | ((ulo + rnd) >> 16))
    return jax.lax.bitcast_convert_type(word, _f32)


def _unpack_bf16(w):
    u = jax.lax.bitcast_convert_type(w, _u32)
    lo = jax.lax.bitcast_convert_type(u << 16, _f32)
    hi = jax.lax.bitcast_convert_type(u & _u32(0xFFFF0000), _f32)
    return jnp.concatenate([lo, hi], axis=1)


def _mesh():
    return plsc.VectorSubcoreMesh(core_axis_name="c", subcore_axis_name="s",
                                  num_cores=NC, num_subcores=NS)


def _prep_body(h_ref, xp_ref, win_ref, bin_ref, a_ref, b_ref,
               h1_ref, ts_ref, td_ref):
    h1 = jnp.dot(h_ref[...], win_ref[...],
                 preferred_element_type=_f32) + bin_ref[...]
    h1_ref[...] = h1
    xp = xp_ref[...]
    p = jnp.dot(h1, a_ref[...], preferred_element_type=_f32)
    q = jnp.dot(h1, b_ref[...], preferred_element_type=_f32)
    ts_ref[...] = jnp.concatenate([_pack_bf16(p), xp], axis=1)
    td_ref[...] = jnp.concatenate([_pack_bf16(q), xp], axis=1)


def _tc_prep(h, xpad, w_in, b_in, a0, b0):
    grid = (N // BN,)
    blk = lambda i: (i, 0)
    full = lambda i: (0, 0)
    return pl.pallas_call(
        _prep_body,
        grid=grid,
        in_specs=[
            pl.BlockSpec((BN, H), blk),
            pl.BlockSpec((BN, HH), blk),
            pl.BlockSpec((H, H), full),
            pl.BlockSpec((1, H), full),
            pl.BlockSpec((H, H), full),
            pl.BlockSpec((H, H), full),
        ],
        out_specs=[
            pl.BlockSpec((BN, H), blk),
            pl.BlockSpec((BN, H), blk),
            pl.BlockSpec((BN, H), blk),
        ],
        out_shape=[
            jax.ShapeDtypeStruct((N, H), _f32),
            jax.ShapeDtypeStruct((N, H), _f32),
            jax.ShapeDtypeStruct((N, H), _f32),
        ],
        compiler_params=pltpu.CompilerParams(
            dimension_semantics=("arbitrary",)),
    )(h, xpad, w_in, b_in, a0, b0)


def _edge0_body(gs_ref, gd_ref, ea_ref, wr_ref, wea_ref, be1_ref,
                we2_ref, be2_ref, wx1_ref, bx1_ref, wx2t_ref, bx2_ref,
                ones_ref, m_ref, t_ref):
    gs = gs_ref[...]
    gd = gd_ref[...]
    ps = _unpack_bf16(gs[:, :HH])
    qd = _unpack_bf16(gd[:, :HH])
    diff = gs[:, HH:] - gd[:, HH:]
    r2b = jnp.dot(diff * diff, ones_ref[...], preferred_element_type=_f32,
                  precision=jax.lax.Precision.HIGHEST)
    a = (ps + qd + r2b * wr_ref[...] +
         jnp.dot(ea_ref[...], wea_ref[...], preferred_element_type=_f32) +
         be1_ref[...])
    m = _silu(jnp.dot(_silu(a), we2_ref[...],
                      preferred_element_type=_f32) + be2_ref[...])
    u = _silu(jnp.dot(m, wx1_ref[...],
                      preferred_element_type=_f32) + bx1_ref[...])
    wb = jnp.dot(u, wx2t_ref[...], preferred_element_type=_f32) + bx2_ref[...]
    m_ref[...] = m
    t_ref[...] = jnp.concatenate(
        [jnp.zeros((gs.shape[0], HH), _f32), diff * wb], axis=1)


def _tc_edge0(gs, gd, ea, wr, wea, be1, we2, be2, wx1, bx1, wx2t, bx2,
              ones_c):
    grid = (EH // BE,)
    blk = lambda i: (i, 0)
    full = lambda i: (0, 0)
    return pl.pallas_call(
        _edge0_body,
        grid=grid,
        in_specs=[
            pl.BlockSpec((BE, H), blk),
            pl.BlockSpec((BE, H), blk),
            pl.BlockSpec((BE, DE), blk),
            pl.BlockSpec((1, H), full),
            pl.BlockSpec((DE, H), full),
            pl.BlockSpec((1, H), full),
            pl.BlockSpec((H, H), full),
            pl.BlockSpec((1, H), full),
            pl.BlockSpec((H, H), full),
            pl.BlockSpec((1, H), full),
            pl.BlockSpec((H, HH), full),
            pl.BlockSpec((1, 1), full),
            pl.BlockSpec((HH, H), full),
        ],
        out_specs=[
            pl.BlockSpec((BE, H), blk),
            pl.BlockSpec((BE, H), blk),
        ],
        out_shape=[
            jax.ShapeDtypeStruct((EH, H), _f32),
            jax.ShapeDtypeStruct((EH, H), _f32),
        ],
        compiler_params=pltpu.CompilerParams(
            dimension_semantics=("arbitrary",)),
    )(gs, gd, ea, wr, wea, be1, we2, be2, wx1, bx1, wx2t, bx2, ones_c)


def _edge1_body(gs_ref, gd_ref, ea_ref, wr_ref, wea_ref, be1_ref,
                we2_ref, be2_ref, ones_ref, m_ref):
    gs = gs_ref[...]
    gd = gd_ref[...]
    ps = _unpack_bf16(gs[:, :HH])
    qd = _unpack_bf16(gd[:, :HH])
    diff = gs[:, HH:] - gd[:, HH:]
    r2b = jnp.dot(diff * diff, ones_ref[...], preferred_element_type=_f32,
                  precision=jax.lax.Precision.HIGHEST)
    a = (ps + qd + r2b * wr_ref[...] +
         jnp.dot(ea_ref[...], wea_ref[...], preferred_element_type=_f32) +
         be1_ref[...])
    m_ref[...] = _silu(jnp.dot(_silu(a), we2_ref[...],
                               preferred_element_type=_f32) + be2_ref[...])


def _tc_edge1(gs, gd, ea, wr, wea, be1, we2, be2, ones_c):
    grid = (EH // BE,)
    blk = lambda i: (i, 0)
    full = lambda i: (0, 0)
    return pl.pallas_call(
        _edge1_body,
        grid=grid,
        in_specs=[
            pl.BlockSpec((BE, H), blk),
            pl.BlockSpec((BE, H), blk),
            pl.BlockSpec((BE, DE), blk),
            pl.BlockSpec((1, H), full),
            pl.BlockSpec((DE, H), full),
            pl.BlockSpec((1, H), full),
            pl.BlockSpec((H, H), full),
            pl.BlockSpec((1, H), full),
            pl.BlockSpec((HH, H), full),
        ],
        out_specs=[pl.BlockSpec((BE, H), blk)],
        out_shape=[jax.ShapeDtypeStruct((EH, H), _f32)],
        compiler_params=pltpu.CompilerParams(
            dimension_semantics=("arbitrary",)),
    )(gs, gd, ea, wr, wea, be1, we2, be2, ones_c)[0]


def _node0_body(h1_ref, xp_ref, aggha_ref, agghb_ref, aggxa_ref, aggxb_ref,
                wh1a_ref, wh1b_ref, bh1_ref, wh2_ref, bh2_ref, a1_ref, b1_ref,
                h2_ref, ts_ref, td_ref):
    h1 = h1_ref[...]
    aggh = ((aggha_ref[0] + aggha_ref[1]) + (agghb_ref[0] + agghb_ref[1]))
    pre = (jnp.dot(h1, wh1a_ref[...], preferred_element_type=_f32) +
           jnp.dot(aggh, wh1b_ref[...], preferred_element_type=_f32) +
           bh1_ref[...])
    upd = jnp.dot(_silu(pre), wh2_ref[...],
                  preferred_element_type=_f32) + bh2_ref[...]
    h2 = h1 + upd
    h2_ref[...] = h2
    axa = aggxa_ref[...]
    axb = aggxb_ref[...]
    aggx = ((axa[0, :, HH:] + axa[1, :, HH:]) +
            (axb[0, :, HH:] + axb[1, :, HH:]))
    xn = xp_ref[...] + aggx * (1.0 / AVG_DEG)
    p = jnp.dot(h2, a1_ref[...], preferred_element_type=_f32)
    q = jnp.dot(h2, b1_ref[...], preferred_element_type=_f32)
    ts_ref[...] = jnp.concatenate([_pack_bf16(p), xn], axis=1)
    td_ref[...] = jnp.concatenate([_pack_bf16(q), xn], axis=1)


def _tc_node0(h1, xpad, aggha, agghb, aggxa, aggxb,
              wh1a, wh1b, bh1, wh2, bh2, a1, b1):
    grid = (N // BN,)
    blk = lambda i: (i, 0)
    blk3c = lambda i: (0, i, 0)
    full = lambda i: (0, 0)
    return pl.pallas_call(
        _node0_body,
        grid=grid,
        in_specs=[
            pl.BlockSpec((BN, H), blk),
            pl.BlockSpec((BN, HH), blk),
            pl.BlockSpec((NC, BN, H), blk3c),
            pl.BlockSpec((NC, BN, H), blk3c),
            pl.BlockSpec((NC, BN, H), blk3c),
            pl.BlockSpec((NC, BN, H), blk3c),
            pl.BlockSpec((H, H), full),
            pl.BlockSpec((H, H), full),
            pl.BlockSpec((1, H), full),
            pl.BlockSpec((H, H), full),
            pl.BlockSpec((1, H), full),
            pl.BlockSpec((H, H), full),
            pl.BlockSpec((H, H), full),
        ],
        out_specs=[
            pl.BlockSpec((BN, H), blk),
            pl.BlockSpec((BN, H), blk),
            pl.BlockSpec((BN, H), blk),
        ],
        out_shape=[
            jax.ShapeDtypeStruct((N, H), _f32),
            jax.ShapeDtypeStruct((N, H), _f32),
            jax.ShapeDtypeStruct((N, H), _f32),
        ],
        compiler_params=pltpu.CompilerParams(
            dimension_semantics=("arbitrary",)),
    )(h1, xpad, aggha, agghb, aggxa, aggxb, wh1a, wh1b, bh1, wh2, bh2, a1, b1)


def _node1_body(h2_ref, aggha_ref, agghb_ref, wh1a_ref, wh1b_ref, bh1_ref,
                wh2_ref, bh2_ref, wout_ref, bout_ref, wfin_ref, bfin_ref,
                out_ref):
    h2 = h2_ref[...]
    aggh = ((aggha_ref[0] + aggha_ref[1]) + (agghb_ref[0] + agghb_ref[1]))
    pre = (jnp.dot(h2, wh1a_ref[...], preferred_element_type=_f32) +
           jnp.dot(aggh, wh1b_ref[...], preferred_element_type=_f32) +
           bh1_ref[...])
    upd = jnp.dot(_silu(pre), wh2_ref[...],
                  preferred_element_type=_f32) + bh2_ref[...]
    hf = h2 + upd
    ho = jnp.dot(hf, wout_ref[...], preferred_element_type=_f32) + bout_ref[...]
    out_ref[...] = jnp.dot(ho, wfin_ref[...],
                           preferred_element_type=_f32) + bfin_ref[...]


def _tc_node1(h2, aggha, agghb, wh1a, wh1b, bh1, wh2, bh2, wout, bout,
              wfinr, bfin):
    grid = (N // BN,)
    blk = lambda i: (i, 0)
    blk3c = lambda i: (0, i, 0)
    full = lambda i: (0, 0)
    return pl.pallas_call(
        _node1_body,
        grid=grid,
        in_specs=[
            pl.BlockSpec((BN, H), blk),
            pl.BlockSpec((NC, BN, H), blk3c),
            pl.BlockSpec((NC, BN, H), blk3c),
            pl.BlockSpec((H, H), full),
            pl.BlockSpec((H, H), full),
            pl.BlockSpec((1, H), full),
            pl.BlockSpec((H, H), full),
            pl.BlockSpec((1, H), full),
            pl.BlockSpec((H, H), full),
            pl.BlockSpec((1, H), full),
            pl.BlockSpec((H, 1), full),
            pl.BlockSpec((1, 1), full),
        ],
        out_specs=[pl.BlockSpec((BN, 1), blk)],
        out_shape=[jax.ShapeDtypeStruct((N, 1), _f32)],
        compiler_params=pltpu.CompilerParams(
            dimension_semantics=("arbitrary",)),
    )(h2, aggha, agghb, wh1a, wh1b, bh1, wh2, bh2, wout, bout, wfinr,
      bfin)[0]


def _sc_gather(ts, td, src2d, dst2d):
    @functools.partial(
        pl.kernel,
        out_type=(jax.ShapeDtypeStruct((EH, H), _f32),
                  jax.ShapeDtypeStruct((EH, H), _f32)),
        mesh=_mesh(),
        scratch_types=[
            pltpu.VMEM((RPT, 1, CHUNK), jnp.int32),
            pltpu.VMEM((RPT, 1, CHUNK), jnp.int32),
            pltpu.VMEM((CHUNK, H), _f32),
            pltpu.VMEM((CHUNK, H), _f32),
            pltpu.VMEM((CHUNK, H), _f32),
            pltpu.VMEM((CHUNK, H), _f32),
            pltpu.SemaphoreType.DMA,
            pltpu.SemaphoreType.DMA,
            pltpu.SemaphoreType.DMA,
            pltpu.SemaphoreType.DMA,
            pltpu.SemaphoreType.DMA,
            pltpu.SemaphoreType.DMA,
            pltpu.SemaphoreType.DMA,
            pltpu.SemaphoreType.DMA,
        ],
    )
    def k(ts_hbm, td_hbm, src_hbm, dst_hbm, gs_hbm, gd_hbm,
          idxs_v, idxd_v, gs0, gd0, gs1, gd1,
          sgs0, sgd0, sgs1, sgd1, sws0, swd0, sws1, swd1):
        cid = jax.lax.axis_index("c")
        sid = jax.lax.axis_index("s")
        wid = sid * NC + cid
        row0 = wid * RPT
        npairs = jnp.where(wid == NW - 1, RPT_LAST // 2, RPT // 2)

        @pl.when(wid < NW - 1)
        def _():
            c1 = pltpu.async_copy(src_hbm.at[pl.ds(row0, RPT)], idxs_v, sgs0)
            c2 = pltpu.async_copy(dst_hbm.at[pl.ds(row0, RPT)], idxd_v, sgd0)
            c1.wait()
            c2.wait()

        @pl.when(wid == NW - 1)
        def _():
            c1 = pltpu.async_copy(src_hbm.at[pl.ds(row0, RPT_LAST)],
                                  idxs_v.at[pl.ds(0, RPT_LAST)], sgs0)
            c2 = pltpu.async_copy(dst_hbm.at[pl.ds(row0, RPT_LAST)],
                                  idxd_v.at[pl.ds(0, RPT_LAST)], sgd0)
            c1.wait()
            c2.wait()

        @pl.loop(0, npairs)
        def _(t):
            ca = 2 * t
            cb = 2 * t + 1
            ga_s = pltpu.async_copy(ts_hbm.at[idxs_v.at[ca, 0]], gs0, sgs0)
            ga_d = pltpu.async_copy(td_hbm.at[idxd_v.at[ca, 0]], gd0, sgd0)
            gb_s = pltpu.async_copy(ts_hbm.at[idxs_v.at[cb, 0]], gs1, sgs1)
            gb_d = pltpu.async_copy(td_hbm.at[idxd_v.at[cb, 0]], gd1, sgd1)
            ga_s.wait()
            wa_s = pltpu.async_copy(
                gs0, gs_hbm.at[pl.ds((row0 + ca) * CHUNK, CHUNK)], sws0)
            ga_d.wait()
            wa_d = pltpu.async_copy(
                gd0, gd_hbm.at[pl.ds((row0 + ca) * CHUNK, CHUNK)], swd0)
            gb_s.wait()
            wb_s = pltpu.async_copy(
                gs1, gs_hbm.at[pl.ds((row0 + cb) * CHUNK, CHUNK)], sws1)
            gb_d.wait()
            wb_d = pltpu.async_copy(
                gd1, gd_hbm.at[pl.ds((row0 + cb) * CHUNK, CHUNK)], swd1)
            wa_s.wait()
            wa_d.wait()
            wb_s.wait()
            wb_d.wait()

    return k(ts, td, src2d, dst2d)


def _sc_scatter(m, src, zh):
    @functools.partial(
        pl.kernel,
        out_type=jax.ShapeDtypeStruct((NC, N, H), _f32),
        mesh=_mesh(),
        scratch_types=[
            pltpu.VMEM((RPT, 1, CHUNK), jnp.int32),
            pltpu.VMEM((CHUNK, H), _f32),
            pltpu.VMEM((CHUNK, H), _f32),
            pltpu.VMEM_SHARED((N, H), _f32),
            pltpu.SemaphoreType.DMA,
            pltpu.SemaphoreType.DMA,
            pltpu.SemaphoreType.DMA,
            pltpu.SemaphoreType.DMA,
        ],
    )
    def k(m_hbm, src_hbm, zh_hbm, aggh_hbm, idx_v, mb0, mb1, aggh_s,
          sl0, sl1, ss0, ss1):
        cid = jax.lax.axis_index("c")
        sid = jax.lax.axis_index("s")
        wid = sid * NC + cid
        row0 = wid * RPT
        npairs = jnp.where(wid == NW - 1, RPT_LAST // 2, RPT // 2)

        @pl.when(wid < NW - 1)
        def _():
            pltpu.async_copy(src_hbm.at[pl.ds(row0, RPT)], idx_v, sl0).wait()

        @pl.when(wid == NW - 1)
        def _():
            pltpu.async_copy(src_hbm.at[pl.ds(row0, RPT_LAST)],
                             idx_v.at[pl.ds(0, RPT_LAST)], sl0).wait()

        r0 = sid * NPT
        pltpu.sync_copy(zh_hbm.at[pl.ds(r0, NPT)], aggh_s.at[pl.ds(r0, NPT)])

        @pl.when(sid == 0)
        def _():
            pltpu.sync_copy(zh_hbm.at[pl.ds(NS * NPT, NREM)],
                            aggh_s.at[pl.ds(NS * NPT, NREM)])

        plsc.subcore_barrier()

        @pl.loop(0, npairs)
        def _(t):
            ca = 2 * t
            cb = 2 * t + 1
            la = pltpu.async_copy(
                m_hbm.at[pl.ds((row0 + ca) * CHUNK, CHUNK)], mb0, sl0)
            lb = pltpu.async_copy(
                m_hbm.at[pl.ds((row0 + cb) * CHUNK, CHUNK)], mb1, sl1)
            la.wait()
            sa = pltpu.async_copy(mb0, aggh_s.at[idx_v.at[ca, 0]], ss0, add=True)
            lb.wait()
            sb = pltpu.async_copy(mb1, aggh_s.at[idx_v.at[cb, 0]], ss1, add=True)
            sa.wait()
            sb.wait()

        plsc.subcore_barrier()
        pltpu.sync_copy(aggh_s.at[pl.ds(r0, NPT)],
                        aggh_hbm.at[cid, pl.ds(r0, NPT)])

        @pl.when(sid == 0)
        def _():
            pltpu.sync_copy(aggh_s.at[pl.ds(NS * NPT, NREM)],
                            aggh_hbm.at[cid, pl.ds(NS * NPT, NREM)])

    return k(m, src, zh)


def kernel(h, x, edge_index, edge_attr, W_in, b_in, We1, be1, We2, be2,
           Wx1, bx1, Wx2, bx2, Wh1, bh1, Wh2, bh2, W_out, b_out, W_fin, b_fin):
    src = edge_index[0].reshape(NCHUNK, 1, CHUNK)
    dst = edge_index[1].reshape(NCHUNK, 1, CHUNK)
    xpad = jnp.pad(x, ((0, 0), (0, HH - x.shape[1])))

    a_l = [We1[l][:H] for l in range(2)]
    b_l = [We1[l][H:2 * H] for l in range(2)]
    wr_l = [We1[l][2 * H:2 * H + 1] for l in range(2)]
    wea_l = [We1[l][2 * H + 1:] for l in range(2)]
    be1_l = [be1[l].reshape(1, H) for l in range(2)]
    be2_l = [be2[l].reshape(1, H) for l in range(2)]
    bx1_0 = bx1[0].reshape(1, H)
    wx2t_0 = jnp.broadcast_to(Wx2[0], (H, HH))
    bx2_0 = bx2[0].reshape(1, 1)
    wh1a_l = [Wh1[l][:H] for l in range(2)]
    wh1b_l = [Wh1[l][H:] for l in range(2)]
    bh1_l = [bh1[l].reshape(1, H) for l in range(2)]
    bh2_l = [bh2[l].reshape(1, H) for l in range(2)]
    b_in2 = b_in.reshape(1, H)
    b_out2 = b_out.reshape(1, H)
    wfin_r = W_fin
    bfin2 = b_fin.reshape(1, 1)
    zh = jnp.zeros((N, H), _f32)
    ones_c = jnp.ones((HH, H), _f32)

    srcA, srcB = src[:NROWS_H], src[NROWS_H:]
    dstA, dstB = dst[:NROWS_H], dst[NROWS_H:]
    eaA, eaB = edge_attr[:EH], edge_attr[EH:]

    h1, ts0, td0 = _tc_prep(h, xpad, W_in, b_in2, a_l[0], b_l[0])
    gsA, gdA = _sc_gather(ts0, td0, srcA, dstA)
    mA, tA = _tc_edge0(gsA, gdA, eaA, wr_l[0], wea_l[0], be1_l[0],
                       We2[0], be2_l[0], Wx1[0], bx1_0, wx2t_0, bx2_0, ones_c)
    gsB, gdB = _sc_gather(ts0, td0, srcB, dstB)
    mB, tB = _tc_edge0(gsB, gdB, eaB, wr_l[0], wea_l[0], be1_l[0],
                       We2[0], be2_l[0], Wx1[0], bx1_0, wx2t_0, bx2_0, ones_c)
    agghA = _sc_scatter(mA, srcA, zh)
    aggxA = _sc_scatter(tA, srcA, zh)
    agghB = _sc_scatter(mB, srcB, zh)
    aggxB = _sc_scatter(tB, srcB, zh)
    h2, ts1, td1 = _tc_node0(h1, xpad, agghA, agghB, aggxA, aggxB,
                             wh1a_l[0], wh1b_l[0], bh1_l[0], Wh2[0], bh2_l[0],
                             a_l[1], b_l[1])

    gs1A, gd1A = _sc_gather(ts1, td1, srcA, dstA)
    m1A = _tc_edge1(gs1A, gd1A, eaA, wr_l[1], wea_l[1], be1_l[1],
                    We2[1], be2_l[1], ones_c)
    gs1B, gd1B = _sc_gather(ts1, td1, srcB, dstB)
    m1B = _tc_edge1(gs1B, gd1B, eaB, wr_l[1], wea_l[1], be1_l[1],
                    We2[1], be2_l[1], ones_c)
    aggh1A = _sc_scatter(m1A, srcA, zh)
    aggh1B = _sc_scatter(m1B, srcB, zh)
    out = _tc_node1(h2, aggh1A, aggh1B, wh1a_l[1], wh1b_l[1], bh1_l[1],
                    Wh2[1], bh2_l[1], W_out, b_out2, wfin_r, bfin2)
    return out
